# baseline (speedup 1.0000x reference)
import functools

import jax
import jax.numpy as jnp
from jax.experimental import pallas as pl

N = 4096
D = 256
BM = 512


def _small_matmul_kernel(x_ref, w_ref, o_ref):
    o_ref[...] = jnp.dot(
        x_ref[...].astype(jnp.bfloat16),
        w_ref[...].astype(jnp.bfloat16),
        preferred_element_type=jnp.float32,
    )


def _adj_matmul_kernel(adj_ref, s_ref, b_ref, o_ref, *, relu):
    t = jnp.dot(
        adj_ref[...].astype(jnp.bfloat16),
        s_ref[...],
        preferred_element_type=jnp.float32,
    )
    y = t + b_ref[...]
    if relu:
        y = jnp.maximum(y, 0.0)
    o_ref[...] = y


def _small_matmul(x, w):
    return pl.pallas_call(
        _small_matmul_kernel,
        out_shape=jax.ShapeDtypeStruct((N, D), jnp.float32),
    )(x, w)


def _adj_matmul(adj, s_bf16, b, relu):
    grid = (N // BM,)
    return pl.pallas_call(
        functools.partial(_adj_matmul_kernel, relu=relu),
        grid=grid,
        in_specs=[
            pl.BlockSpec((BM, N), lambda i: (i, 0)),
            pl.BlockSpec((N, D), lambda i: (0, 0)),
            pl.BlockSpec((1, D), lambda i: (0, 0)),
        ],
        out_specs=pl.BlockSpec((BM, D), lambda i: (i, 0)),
        out_shape=jax.ShapeDtypeStruct((N, D), jnp.float32),
    )(adj, s_bf16, b)


def kernel(x, adj, W1, b1, W2, b2):
    b1r = b1.reshape(1, D)
    b2r = b2.reshape(1, D)
    s1 = _small_matmul(x, W1).astype(jnp.bfloat16)
    h = _adj_matmul(adj, s1, b1r, relu=True)
    s2 = _small_matmul(h, W2).astype(jnp.bfloat16)
    out = _adj_matmul(adj, s2, b2r, relu=False)
    return out


# speedup vs baseline: 1.4985x; 1.4985x over previous
import jax
import jax.numpy as jnp
from jax.experimental import pallas as pl
from jax.experimental.pallas import tpu as pltpu

N = 4096
D = 256
BM = 512
NB = N // BM


def _fused_gcn_kernel(adj_ref, x_ref, w1_ref, b1_ref, w2_ref, b2_ref,
                      o_ref, adjbf_ref, s_ref, h_ref):
    i = pl.program_id(0)

    @pl.when(i == 0)
    def _():
        s_ref[...] = jnp.dot(
            x_ref[...], w1_ref[...], preferred_element_type=jnp.float32
        ).astype(jnp.bfloat16)

    @pl.when(i < NB)
    def _():
        ab = adj_ref[...].astype(jnp.bfloat16)
        adjbf_ref[pl.ds(i * BM, BM), :] = ab
        t = jnp.dot(ab, s_ref[...], preferred_element_type=jnp.float32)
        h_ref[pl.ds(i * BM, BM), :] = jnp.maximum(
            t + b1_ref[...], 0.0
        ).astype(jnp.bfloat16)

    @pl.when(i == NB)
    def _():
        s_ref[...] = jnp.dot(
            h_ref[...], w2_ref[...], preferred_element_type=jnp.float32
        ).astype(jnp.bfloat16)

    @pl.when(i >= NB)
    def _():
        ab = adjbf_ref[pl.ds((i - NB) * BM, BM), :]
        o_ref[...] = (
            jnp.dot(ab, s_ref[...], preferred_element_type=jnp.float32)
            + b2_ref[...]
        )


def kernel(x, adj, W1, b1, W2, b2):
    xb = x.astype(jnp.bfloat16)
    w1b = W1.astype(jnp.bfloat16)
    w2b = W2.astype(jnp.bfloat16)
    b1r = b1.reshape(1, D)
    b2r = b2.reshape(1, D)
    return pl.pallas_call(
        _fused_gcn_kernel,
        grid=(2 * NB,),
        in_specs=[
            pl.BlockSpec((BM, N), lambda i: (jnp.minimum(i, NB - 1), 0)),
            pl.BlockSpec((N, D), lambda i: (0, 0)),
            pl.BlockSpec((D, D), lambda i: (0, 0)),
            pl.BlockSpec((1, D), lambda i: (0, 0)),
            pl.BlockSpec((D, D), lambda i: (0, 0)),
            pl.BlockSpec((1, D), lambda i: (0, 0)),
        ],
        out_specs=pl.BlockSpec((BM, D), lambda i: (jnp.maximum(i - NB, 0), 0)),
        out_shape=jax.ShapeDtypeStruct((N, D), jnp.float32),
        scratch_shapes=[
            pltpu.VMEM((N, N), jnp.bfloat16),
            pltpu.VMEM((N, D), jnp.bfloat16),
            pltpu.VMEM((N, D), jnp.bfloat16),
        ],
    )(adj, xb, w1b, b1r, w2b, b2r)
